# baseline (speedup 1.0000x reference)
import functools

import jax
import jax.numpy as jnp
from jax import lax
from jax.experimental import pallas as pl
from jax.experimental.pallas import tpu as pltpu
from jax.experimental.pallas import tpu_sc as plsc

B = 16384
D = 32
GW = 128
NC, NS, L = 2, 16, 16
NW = NC * NS
BPW = B // NW
NCH = 4
CHUNK = BPW // NCH
GPC = CHUNK // L
NWROWS = 2 * D + 1


def _body(ru_hbm, ri_hbm, ou_hbm, oi_hbm, ut_hbm, it_hbm, w_hbm, out_hbm,
          idx_u, idx_i, offs_u, offs_i, u_buf, i_buf, w_v, out_v,
          sem_u0, sem_u1, sem_i0, sem_i1):
    wid = lax.axis_index("s") * NC + lax.axis_index("c")
    base = wid * BPW
    for j in range(NCH):
        pltpu.sync_copy(ru_hbm.at[pl.ds(base + j * CHUNK, CHUNK)], idx_u.at[j])
        pltpu.sync_copy(ri_hbm.at[pl.ds(base + j * CHUNK, CHUNK)], idx_i.at[j])
    pltpu.sync_copy(ou_hbm.at[pl.ds(base, BPW)], offs_u)
    pltpu.sync_copy(oi_hbm.at[pl.ds(base, BPW)], offs_i)
    pltpu.sync_copy(w_hbm, w_v)

    sems_u = (sem_u0, sem_u1)
    sems_i = (sem_i0, sem_i1)

    def start(j):
        k = j % 2
        cu = pltpu.async_copy(ut_hbm.at[idx_u.at[j]], u_buf.at[k], sems_u[k])
        ci = pltpu.async_copy(it_hbm.at[idx_i.at[j]], i_buf.at[k], sems_i[k])
        return cu, ci

    lanes = lax.iota(jnp.int32, L)
    inflight = start(0)
    for j in range(NCH):
        cu, ci = inflight
        if j + 1 < NCH:
            nxt = start(j + 1)
        cu.wait()
        ci.wait()
        if j + 1 < NCH:
            inflight = nxt
        k = j % 2
        kvec = jnp.full((L,), k, dtype=jnp.int32)

        def group(g, carry):
            rows = g * L + lanes
            gb = j * CHUNK + g * L
            ou = offs_u[pl.ds(gb, L)]
            oi = offs_i[pl.ds(gb, L)]
            acc = w_v[pl.ds(2 * D * L, L)]
            for d in range(D):
                acc = acc + plsc.load_gather(
                    u_buf, [kvec, rows, ou + d]) * w_v[pl.ds(d * L, L)]
                acc = acc + plsc.load_gather(
                    i_buf, [kvec, rows, oi + d]) * w_v[pl.ds((D + d) * L, L)]
            out_v[pl.ds(gb, L)] = acc
            return carry

        lax.fori_loop(0, GPC, group, 0)
    pltpu.sync_copy(out_v, out_hbm.at[pl.ds(base, BPW)])


_mesh = plsc.VectorSubcoreMesh(core_axis_name="c", subcore_axis_name="s")

_ncf = functools.partial(
    pl.kernel, mesh=_mesh,
    compiler_params=pltpu.CompilerParams(
        needs_layout_passes=False, use_tc_tiling_on_sc=True),
    out_type=jax.ShapeDtypeStruct((B,), jnp.float32),
    scratch_types=[
        pltpu.VMEM((NCH, CHUNK), jnp.int32),
        pltpu.VMEM((NCH, CHUNK), jnp.int32),
        pltpu.VMEM((BPW,), jnp.int32),
        pltpu.VMEM((BPW,), jnp.int32),
        pltpu.VMEM((2, CHUNK, GW), jnp.float32),
        pltpu.VMEM((2, CHUNK, GW), jnp.float32),
        pltpu.VMEM((NWROWS * L,), jnp.float32),
        pltpu.VMEM((BPW,), jnp.float32),
        pltpu.SemaphoreType.DMA,
        pltpu.SemaphoreType.DMA,
        pltpu.SemaphoreType.DMA,
        pltpu.SemaphoreType.DMA,
    ],
)(_body)


def kernel(user_ids, item_ids, user_table, item_table, fc_w, fc_b):
    uid = user_ids.astype(jnp.int32)
    iid = item_ids.astype(jnp.int32)
    rpg = GW // D
    w_all = jnp.repeat(
        jnp.concatenate([fc_w.reshape(-1), fc_b.reshape(-1)]).astype(jnp.float32),
        L,
    )
    out = _ncf(
        uid // rpg, iid // rpg,
        (uid % rpg) * D, (iid % rpg) * D,
        user_table.reshape(-1, GW), item_table.reshape(-1, GW),
        w_all,
    )
    return out.reshape(B, 1)
